# baseline (speedup 1.0000x reference)
import functools

import jax
import jax.numpy as jnp
from jax import lax
from jax.experimental import pallas as pl
from jax.experimental.pallas import tpu as pltpu
from jax.experimental.pallas import tpu_sc as plsc

D_MODEL = 64
SCALE = 8.0

NUM_CORES = 2
NUM_SUBCORES = 16
NUM_WORKERS = NUM_CORES * NUM_SUBCORES

CHUNK = 128


@functools.lru_cache(maxsize=None)
def _build(B):
    assert B % (NUM_WORKERS * CHUNK) == 0
    n_chunks = B // (NUM_WORKERS * CHUNK)

    mesh = plsc.VectorSubcoreMesh(
        core_axis_name="c",
        subcore_axis_name="s",
        num_cores=NUM_CORES,
        num_subcores=NUM_SUBCORES,
    )

    @functools.partial(
        pl.kernel,
        out_type=jax.ShapeDtypeStruct((B, D_MODEL), jnp.float32),
        mesh=mesh,
        scratch_types=[
            pltpu.VMEM((n_chunks, CHUNK), jnp.int32),
            pltpu.VMEM((CHUNK, D_MODEL), jnp.float32),
            pltpu.SemaphoreType.DMA,
        ],
        compiler_params=pltpu.CompilerParams(use_tc_tiling_on_sc=False),
    )
    def emb_kernel(idx_hbm, table_hbm, out_hbm, idx_v, rows_v, sem):
        wid = lax.axis_index("s") * NUM_CORES + lax.axis_index("c")
        pltpu.sync_copy(idx_hbm.at[pl.ds(wid * n_chunks, n_chunks)], idx_v)
        w_base = wid * n_chunks * CHUNK

        @pl.loop(0, n_chunks)
        def _chunk(c):
            pltpu.async_copy(table_hbm.at[idx_v.at[c]], rows_v, sem).wait()

            @pl.loop(0, CHUNK, unroll=4)
            def _scale(i):
                for k in range(D_MODEL // 16):
                    sl = pl.ds(k * 16, 16)
                    rows_v[i, sl] = rows_v[i, sl] * SCALE

            pltpu.sync_copy(
                rows_v, out_hbm.at[pl.ds(w_base + c * CHUNK, CHUNK)]
            )

    return emb_kernel


def kernel(x, lut):
    b, s = x.shape
    B = b * s
    idx = x.reshape(B // CHUNK, CHUNK)
    out = _build(B)(idx, lut)
    return out.reshape(b, s, D_MODEL)
